# Initial kernel scaffold; baseline (speedup 1.0000x reference)
import jax
import jax.numpy as jnp
from jax.experimental import pallas as pl


def kernel(images, labels):
    raise NotImplementedError("write your pallas kernel here")


# speedup vs baseline: 1.9986x; 1.9986x over previous
import functools

import jax
import jax.numpy as jnp
from jax import lax
from jax.experimental import pallas as pl
from jax.experimental.pallas import tpu as pltpu
from jax.experimental.pallas import tpu_sc as plsc

_NUM_CLASSES = 10
_B = 16384
_NC, _NS, _L = 2, 16, 16
_NW = _NC * _NS
_BPW = _B // _NW
_CHUNKS = _BPW // _L


@functools.partial(
    pl.kernel,
    out_type=jax.ShapeDtypeStruct((_B * _NUM_CLASSES,), jnp.float32),
    mesh=plsc.VectorSubcoreMesh(core_axis_name="c", subcore_axis_name="s"),
    scratch_types=[
        pltpu.VMEM((_BPW,), jnp.int32),
        pltpu.VMEM((_BPW * _NUM_CLASSES,), jnp.float32),
    ],
    compiler_params=pltpu.CompilerParams(needs_layout_passes=False),
)
def _onehot_sc(labels_hbm, out_hbm, lab_v, out_v):
    wid = lax.axis_index("s") * _NC + lax.axis_index("c")
    base = wid * _BPW
    pltpu.sync_copy(labels_hbm.at[pl.ds(base, _BPW)], lab_v)

    zeros = jnp.zeros((_L,), jnp.float32)
    ones = jnp.ones((_L,), jnp.float32)
    lane = lax.iota(jnp.int32, _L)

    def body(i, carry):
        o = i * (_L * _NUM_CLASSES)
        for k in range(_NUM_CLASSES):
            out_v[pl.ds(o + k * _L, _L)] = zeros
        lab = lab_v[pl.ds(i * _L, _L)]
        idx = (i * _L + lane) * _NUM_CLASSES + lab
        plsc.store_scatter(out_v, [idx], ones)
        return carry

    lax.fori_loop(0, _CHUNKS, body, 0)

    pltpu.sync_copy(
        out_v, out_hbm.at[pl.ds(base * _NUM_CLASSES, _BPW * _NUM_CLASSES)]
    )


def kernel(images, labels):
    del images
    flat = _onehot_sc(labels)
    return flat.reshape(_B, _NUM_CLASSES)


# speedup vs baseline: 2.0519x; 1.0266x over previous
import functools

import jax
import jax.numpy as jnp
from jax import lax
from jax.experimental import pallas as pl
from jax.experimental.pallas import tpu as pltpu
from jax.experimental.pallas import tpu_sc as plsc

_NUM_CLASSES = 10
_B = 16384
_NC, _NS, _L = 1, 16, 16
_NW = _NC * _NS
_BPW = _B // _NW
_CHUNKS = _BPW // _L


@functools.partial(
    pl.kernel,
    out_type=jax.ShapeDtypeStruct((_B * _NUM_CLASSES,), jnp.float32),
    mesh=plsc.VectorSubcoreMesh(
        core_axis_name="c", subcore_axis_name="s", num_cores=_NC
    ),
    scratch_types=[
        pltpu.VMEM((_BPW,), jnp.int32),
        pltpu.VMEM((_BPW * _NUM_CLASSES,), jnp.float32),
    ],
    compiler_params=pltpu.CompilerParams(needs_layout_passes=False),
)
def _onehot_sc(labels_hbm, out_hbm, lab_v, out_v):
    wid = lax.axis_index("s") * _NC + lax.axis_index("c")
    base = wid * _BPW
    pltpu.sync_copy(labels_hbm.at[pl.ds(base, _BPW)], lab_v)

    zeros = jnp.zeros((_L,), jnp.float32)
    ones = jnp.ones((_L,), jnp.float32)
    lane = lax.iota(jnp.int32, _L)

    def body(i, carry):
        o = i * (_L * _NUM_CLASSES)
        for k in range(_NUM_CLASSES):
            out_v[pl.ds(o + k * _L, _L)] = zeros
        lab = lab_v[pl.ds(i * _L, _L)]
        idx = (i * _L + lane) * _NUM_CLASSES + lab
        plsc.store_scatter(out_v, [idx], ones)
        return carry

    lax.fori_loop(0, _CHUNKS, body, 0)

    pltpu.sync_copy(
        out_v, out_hbm.at[pl.ds(base * _NUM_CLASSES, _BPW * _NUM_CLASSES)]
    )


def kernel(images, labels):
    del images
    flat = _onehot_sc(labels)
    return flat.reshape(_B, _NUM_CLASSES)


# speedup vs baseline: 2.6027x; 1.2685x over previous
import functools

import jax
import jax.numpy as jnp
from jax import lax
from jax.experimental import pallas as pl
from jax.experimental.pallas import tpu as pltpu
from jax.experimental.pallas import tpu_sc as plsc

_NUM_CLASSES = 10
_B = 16384
_NC, _NS, _L = 2, 16, 16
_NW = _NC * _NS
_BPW = _B // _NW
_CHUNKS = _BPW // _L


@functools.partial(
    pl.kernel,
    out_type=jax.ShapeDtypeStruct((_B, _NUM_CLASSES), jnp.float32),
    mesh=plsc.VectorSubcoreMesh(
        core_axis_name="c", subcore_axis_name="s", num_cores=_NC
    ),
    scratch_types=[
        pltpu.VMEM((_BPW,), jnp.int32),
        pltpu.VMEM((_BPW, _NUM_CLASSES), jnp.float32),
    ],
    compiler_params=pltpu.CompilerParams(
        needs_layout_passes=False, use_tc_tiling_on_sc=True
    ),
)
def _onehot_sc(labels_hbm, out_hbm, lab_v, out_v):
    wid = lax.axis_index("s") * _NC + lax.axis_index("c")
    base = wid * _BPW
    pltpu.sync_copy(labels_hbm.at[pl.ds(base, _BPW)], lab_v)

    zeros = jnp.zeros((_L,), jnp.float32)
    ones = jnp.ones((_L,), jnp.float32)
    lane = lax.iota(jnp.int32, _L)

    def body(i, carry):
        rows = i * _L + lane
        lab = lab_v[pl.ds(i * _L, _L)]
        for k in range(_NUM_CLASSES):
            col = lab + k
            col = jnp.where(col >= _NUM_CLASSES, col - _NUM_CLASSES, col)
            plsc.store_scatter(out_v, [rows, col], ones if k == 0 else zeros)
        return carry

    lax.fori_loop(0, _CHUNKS, body, 0)

    pltpu.sync_copy(out_v, out_hbm.at[pl.ds(base, _BPW)])


def kernel(images, labels):
    del images
    return _onehot_sc(labels)


# speedup vs baseline: 3.7474x; 1.4398x over previous
import functools

import jax
import jax.numpy as jnp
from jax import lax
from jax.experimental import pallas as pl
from jax.experimental.pallas import tpu as pltpu
from jax.experimental.pallas import tpu_sc as plsc

_NUM_CLASSES = 10
_B = 16384
_NC, _NS, _L = 2, 16, 16
_NW = _NC * _NS
_BPW = _B // _NW
_CHUNKS = _BPW // _L


@functools.partial(
    pl.kernel,
    out_type=jax.ShapeDtypeStruct((_NUM_CLASSES, _B), jnp.float32),
    mesh=plsc.VectorSubcoreMesh(
        core_axis_name="c", subcore_axis_name="s", num_cores=_NC
    ),
    scratch_types=[
        pltpu.VMEM((_BPW,), jnp.int32),
        pltpu.VMEM((_NUM_CLASSES, _BPW), jnp.float32),
    ],
    compiler_params=pltpu.CompilerParams(
        needs_layout_passes=False, use_tc_tiling_on_sc=True
    ),
)
def _onehot_t_sc(labels_hbm, out_hbm, lab_v, out_v):
    wid = lax.axis_index("s") * _NC + lax.axis_index("c")
    base = wid * _BPW
    pltpu.sync_copy(labels_hbm.at[pl.ds(base, _BPW)], lab_v)

    ones = jnp.ones((_L,), jnp.float32)
    zeros = jnp.zeros((_L,), jnp.float32)

    def body(i, carry):
        lab = lab_v[pl.ds(i * _L, _L)]
        for j in range(_NUM_CLASSES):
            out_v[j, pl.ds(i * _L, _L)] = jnp.where(lab == j, ones, zeros)
        return carry

    lax.fori_loop(0, _CHUNKS, body, 0)

    pltpu.sync_copy(out_v, out_hbm.at[:, pl.ds(base, _BPW)])


def kernel(images, labels):
    del images
    return _onehot_t_sc(labels).T


# speedup vs baseline: 3.8530x; 1.0282x over previous
import functools

import jax
import jax.numpy as jnp
from jax import lax
from jax.experimental import pallas as pl
from jax.experimental.pallas import tpu as pltpu
from jax.experimental.pallas import tpu_sc as plsc

_NUM_CLASSES = 10
_B = 16384
_NC, _NS, _L = 1, 16, 16
_NW = _NC * _NS
_BPW = _B // _NW
_CHUNKS = _BPW // _L


@functools.partial(
    pl.kernel,
    out_type=jax.ShapeDtypeStruct((_NUM_CLASSES, _B), jnp.float32),
    mesh=plsc.VectorSubcoreMesh(
        core_axis_name="c", subcore_axis_name="s", num_cores=_NC
    ),
    scratch_types=[
        pltpu.VMEM((_BPW,), jnp.int32),
        pltpu.VMEM((_NUM_CLASSES, _BPW), jnp.float32),
    ],
    compiler_params=pltpu.CompilerParams(
        needs_layout_passes=False, use_tc_tiling_on_sc=True
    ),
)
def _onehot_t_sc(labels_hbm, out_hbm, lab_v, out_v):
    wid = lax.axis_index("s") * _NC + lax.axis_index("c")
    base = wid * _BPW
    pltpu.sync_copy(labels_hbm.at[pl.ds(base, _BPW)], lab_v)

    ones = jnp.ones((_L,), jnp.float32)
    zeros = jnp.zeros((_L,), jnp.float32)

    def body(i, carry):
        lab = lab_v[pl.ds(i * _L, _L)]
        for j in range(_NUM_CLASSES):
            out_v[j, pl.ds(i * _L, _L)] = jnp.where(lab == j, ones, zeros)
        return carry

    lax.fori_loop(0, _CHUNKS, body, 0)

    pltpu.sync_copy(out_v, out_hbm.at[:, pl.ds(base, _BPW)])


def kernel(images, labels):
    del images
    return _onehot_t_sc(labels).T
